# baseline (speedup 1.0000x reference)
import functools

import jax
import jax.numpy as jnp
from jax import lax
from jax.experimental import pallas as pl
from jax.experimental.pallas import tpu as pltpu
from jax.experimental.pallas import tpu_sc as plsc

_N_TOKENS = 65536
_CODE_SIZE = 32
_K = 8192
_TB = 1024


def _argmax_body(x_ref, cb_ref, idx_ref):
    x = x_ref[...]
    n2 = jnp.sum(x * x, axis=-1, keepdims=True)
    e = x / jnp.maximum(jnp.sqrt(n2), 1e-12)
    s = lax.dot_general(e, cb_ref[...], (((1,), (1,)), ((), ())),
                        preferred_element_type=jnp.float32)
    m_run = jnp.full((_TB, 128), -jnp.inf, jnp.float32)
    col = jnp.zeros((_TB, 128), jnp.int32)
    for c in range(_K // 128):
        sc = s[:, c * 128:(c + 1) * 128]
        upd = sc > m_run
        col = jnp.where(upd, c, col)
        m_run = jnp.maximum(sc, m_run)
    m = jnp.max(m_run, axis=1, keepdims=True)
    lane = lax.broadcasted_iota(jnp.int32, (_TB, 128), 1)
    g = jnp.where(m_run == m, col * 128 + lane, _K)
    idx_ref[...] = jnp.min(g, axis=1)


def _tc_argmax(x, cbt):
    return pl.pallas_call(
        _argmax_body,
        grid=(_N_TOKENS // _TB,),
        in_specs=[
            pl.BlockSpec((_TB, _CODE_SIZE), lambda i: (i, 0)),
            pl.BlockSpec((_K, _CODE_SIZE), lambda i: (0, 0)),
        ],
        out_specs=pl.BlockSpec((_TB,), lambda i: (i,)),
        out_shape=jax.ShapeDtypeStruct((_N_TOKENS,), jnp.int32),
    )(x, cbt)


def _sc_gather(codebook, idx3):
    info = plsc.get_sparse_core_info()
    nc, ns = info.num_cores, info.num_subcores
    nw = nc * ns
    bpw = _N_TOKENS // nw
    ch = bpw // 128
    mesh = plsc.VectorSubcoreMesh(core_axis_name="c", subcore_axis_name="s")

    @functools.partial(
        pl.kernel,
        mesh=mesh,
        compiler_params=pltpu.CompilerParams(use_tc_tiling_on_sc=False),
        out_type=jax.ShapeDtypeStruct((_N_TOKENS, _CODE_SIZE), jnp.float32),
        scratch_types=[
            pltpu.VMEM((ch, 128), jnp.int32),
            pltpu.VMEM((bpw, _CODE_SIZE), jnp.float32),
            pltpu.SemaphoreType.DMA,
        ],
    )
    def gather_kernel(table_hbm, idx_hbm, out_hbm, idx_v, rows_v, sem):
        wid = lax.axis_index("s") * nc + lax.axis_index("c")
        base = wid * bpw
        pltpu.sync_copy(idx_hbm.at[wid], idx_v)
        copies = []
        for j in range(ch):
            copies.append(
                pltpu.async_copy(
                    table_hbm.at[idx_v.at[j]],
                    rows_v.at[pl.ds(j * 128, 128)],
                    sem,
                ))
        for c in copies:
            c.wait()
        pltpu.sync_copy(rows_v, out_hbm.at[pl.ds(base, bpw)])

    return gather_kernel(codebook, idx3)


def kernel(x, codebook):
    idx = _tc_argmax(x, codebook)
    info = plsc.get_sparse_core_info()
    nw = info.num_cores * info.num_subcores
    idx3 = idx.reshape(nw, _N_TOKENS // nw // 128, 128)
    return _sc_gather(codebook, idx3)


# speedup vs baseline: 1.2576x; 1.2576x over previous
import functools

import jax
import jax.numpy as jnp
from jax import lax
from jax.experimental import pallas as pl
from jax.experimental.pallas import tpu as pltpu
from jax.experimental.pallas import tpu_sc as plsc

_N_TOKENS = 65536
_CODE_SIZE = 32
_K = 8192
_TB = 1024


def _argmax_body(x_ref, cb_ref, idx_ref):
    x = x_ref[...]
    n2 = jnp.sum(x * x, axis=-1, keepdims=True)
    e = x / jnp.maximum(jnp.sqrt(n2), 1e-12)
    st = lax.dot_general(cb_ref[...], e, (((1,), (1,)), ((), ())),
                         preferred_element_type=jnp.float32)
    ngrp = 4
    rows_per_grp = _K // 8 // ngrp
    sub = lax.broadcasted_iota(jnp.int32, (8, _TB), 0)
    vals, idxs = [], []
    for g in range(ngrp):
        m_run = jnp.full((8, _TB), -jnp.inf, jnp.float32)
        col = jnp.zeros((8, _TB), jnp.int32)
        for r in range(rows_per_grp):
            row = g * rows_per_grp + r
            sc = st[row * 8:(row + 1) * 8, :]
            upd = sc > m_run
            col = jnp.where(upd, row, col)
            m_run = jnp.maximum(sc, m_run)
        vals.append(m_run)
        idxs.append(col * 8 + sub)
    v, i = vals[0], idxs[0]
    for g in range(1, ngrp):
        take = vals[g] > v
        i = jnp.where(take, idxs[g], i)
        v = jnp.maximum(vals[g], v)
    for sh in (4, 2, 1):
        v2 = jnp.concatenate([v[sh:], v[:sh]], axis=0)
        i2 = jnp.concatenate([i[sh:], i[:sh]], axis=0)
        take = (v2 > v) | ((v2 == v) & (i2 < i))
        v = jnp.where(take, v2, v)
        i = jnp.where(take, i2, i)
    idx_ref[...] = i[0, :]


def _tc_argmax(x, cbt):
    return pl.pallas_call(
        _argmax_body,
        grid=(_N_TOKENS // _TB,),
        in_specs=[
            pl.BlockSpec((_TB, _CODE_SIZE), lambda i: (i, 0)),
            pl.BlockSpec((_K, _CODE_SIZE), lambda i: (0, 0)),
        ],
        out_specs=pl.BlockSpec((_TB,), lambda i: (i,)),
        out_shape=jax.ShapeDtypeStruct((_N_TOKENS,), jnp.int32),
    )(x, cbt)


def _sc_gather(codebook, idx3):
    info = plsc.get_sparse_core_info()
    nc, ns = info.num_cores, info.num_subcores
    nw = nc * ns
    bpw = _N_TOKENS // nw
    ch = bpw // 128
    mesh = plsc.VectorSubcoreMesh(core_axis_name="c", subcore_axis_name="s")

    @functools.partial(
        pl.kernel,
        mesh=mesh,
        compiler_params=pltpu.CompilerParams(use_tc_tiling_on_sc=False),
        out_type=jax.ShapeDtypeStruct((_N_TOKENS, _CODE_SIZE), jnp.float32),
        scratch_types=[
            pltpu.VMEM((ch, 128), jnp.int32),
            pltpu.VMEM((bpw, _CODE_SIZE), jnp.float32),
            pltpu.SemaphoreType.DMA,
        ],
    )
    def gather_kernel(table_hbm, idx_hbm, out_hbm, idx_v, rows_v, sem):
        wid = lax.axis_index("s") * nc + lax.axis_index("c")
        base = wid * bpw
        pltpu.sync_copy(idx_hbm.at[wid], idx_v)
        copies = []
        for j in range(ch):
            copies.append(
                pltpu.async_copy(
                    table_hbm.at[idx_v.at[j]],
                    rows_v.at[pl.ds(j * 128, 128)],
                    sem,
                ))
        for c in copies:
            c.wait()
        pltpu.sync_copy(rows_v, out_hbm.at[pl.ds(base, bpw)])

    return gather_kernel(codebook, idx3)


def kernel(x, codebook):
    idx = _tc_argmax(x, codebook)
    info = plsc.get_sparse_core_info()
    nw = info.num_cores * info.num_subcores
    idx3 = idx.reshape(nw, _N_TOKENS // nw // 128, 128)
    return _sc_gather(codebook, idx3)


# speedup vs baseline: 1.2596x; 1.0016x over previous
import functools

import jax
import jax.numpy as jnp
from jax import lax
from jax.experimental import pallas as pl
from jax.experimental.pallas import tpu as pltpu
from jax.experimental.pallas import tpu_sc as plsc

_N_TOKENS = 65536
_CODE_SIZE = 32
_K = 8192
_TB = 1024


def _argmax_body(x_ref, cb_ref, idx_ref):
    x = x_ref[...]
    n2 = jnp.sum(x * x, axis=-1, keepdims=True)
    e = x / jnp.maximum(jnp.sqrt(n2), 1e-12)
    st = lax.dot_general(cb_ref[...], e, (((1,), (1,)), ((), ())),
                         preferred_element_type=jnp.float32)
    ngrp = 4
    rows_per_grp = _K // 8 // ngrp
    sub = lax.broadcasted_iota(jnp.int32, (8, _TB), 0)
    vals, idxs = [], []
    for g in range(ngrp):
        m_run = jnp.full((8, _TB), -jnp.inf, jnp.float32)
        col = jnp.zeros((8, _TB), jnp.int32)
        for r in range(rows_per_grp):
            row = g * rows_per_grp + r
            sc = st[row * 8:(row + 1) * 8, :]
            upd = sc > m_run
            col = jnp.where(upd, row, col)
            m_run = jnp.maximum(sc, m_run)
        vals.append(m_run)
        idxs.append(col * 8 + sub)
    v, i = vals[0], idxs[0]
    for g in range(1, ngrp):
        take = vals[g] > v
        i = jnp.where(take, idxs[g], i)
        v = jnp.maximum(vals[g], v)
    for sh in (4, 2, 1):
        v2 = jnp.concatenate([v[sh:], v[:sh]], axis=0)
        i2 = jnp.concatenate([i[sh:], i[:sh]], axis=0)
        take = (v2 > v) | ((v2 == v) & (i2 < i))
        v = jnp.where(take, v2, v)
        i = jnp.where(take, i2, i)
    idx_ref[...] = i[0, :]


def _tc_argmax(x, cbt):
    return pl.pallas_call(
        _argmax_body,
        grid=(_N_TOKENS // _TB,),
        in_specs=[
            pl.BlockSpec((_TB, _CODE_SIZE), lambda i: (i, 0)),
            pl.BlockSpec((_K, _CODE_SIZE), lambda i: (0, 0)),
        ],
        out_specs=pl.BlockSpec((_TB,), lambda i: (i,)),
        out_shape=jax.ShapeDtypeStruct((_N_TOKENS,), jnp.int32),
    )(x, cbt)


def _sc_gather(codebook, idx3):
    info = plsc.get_sparse_core_info()
    nc, ns = info.num_cores, info.num_subcores
    nw = nc * ns
    bpw = _N_TOKENS // nw
    ch = bpw // 128
    mesh = plsc.VectorSubcoreMesh(core_axis_name="c", subcore_axis_name="s")

    @functools.partial(
        pl.kernel,
        mesh=mesh,
        compiler_params=pltpu.CompilerParams(use_tc_tiling_on_sc=False),
        out_type=jax.ShapeDtypeStruct((nw, bpw, _CODE_SIZE), jnp.float32),
        scratch_types=[
            pltpu.VMEM((ch, 128), jnp.int32),
            pltpu.VMEM((bpw, _CODE_SIZE), jnp.float32),
            pltpu.SemaphoreType.DMA,
        ],
    )
    def gather_kernel(table_hbm, idx_hbm, out_hbm, idx_v, rows_v, sem):
        wid = lax.axis_index("s") * nc + lax.axis_index("c")
        pltpu.sync_copy(idx_hbm.at[wid], idx_v)
        copies = []
        for j in range(ch):
            copies.append(
                pltpu.async_copy(
                    table_hbm.at[idx_v.at[j]],
                    rows_v.at[pl.ds(j * 128, 128)],
                    sem,
                ))
        for c in copies:
            c.wait()
        pltpu.sync_copy(rows_v, out_hbm.at[wid])

    return gather_kernel(codebook, idx3).reshape(_N_TOKENS, _CODE_SIZE)


def kernel(x, codebook):
    idx = _tc_argmax(x, codebook)
    info = plsc.get_sparse_core_info()
    nw = info.num_cores * info.num_subcores
    idx3 = idx.reshape(nw, _N_TOKENS // nw // 128, 128)
    return _sc_gather(codebook, idx3)
